# baseline (speedup 1.0000x reference)
import functools

import jax
import jax.numpy as jnp
from jax import lax
from jax.experimental import pallas as pl
from jax.experimental.pallas import tpu as pltpu
from jax.experimental.pallas import tpu_sc as plsc

_NC, _NS = 2, 16
_NW = _NC * _NS

_IDX_ROW = 128
_K = 8
_CHUNK = _IDX_ROW * _K


def _proj_body(x_ref, w_ref, o_ref):
    o_ref[...] = lax.dot_general(
        x_ref[...], w_ref[...],
        (((1,), (1,)), ((), ())),
        preferred_element_type=jnp.float32,
    )


def _project_table(emb, W):
    V, D = emb.shape
    H = W.shape[0]
    block = 8000
    assert V % block == 0
    return pl.pallas_call(
        _proj_body,
        grid=(V // block,),
        in_specs=[
            pl.BlockSpec((block, D), lambda i: (i, 0)),
            pl.BlockSpec((H, D), lambda i: (0, 0)),
        ],
        out_specs=pl.BlockSpec((block, H), lambda i: (i, 0)),
        out_shape=jax.ShapeDtypeStruct((V, H), jnp.float32),
    )(emb, W)


def _gather_body(n_chunks_per_w, tab_ref, idx1_ref, idx2_ref,
                 out1_ref, out2_ref, idx_v, rows_v, sem):
    wid = lax.axis_index("s") * _NC + lax.axis_index("c")
    H = tab_ref.shape[1]

    for idx_ref, out_ref in ((idx1_ref, out1_ref), (idx2_ref, out2_ref)):
        def body(i, _, idx_ref=idx_ref, out_ref=out_ref):
            row0 = (wid * n_chunks_per_w + i) * _K
            pltpu.sync_copy(idx_ref.at[pl.ds(row0, _K)], idx_v)
            copies = [
                pltpu.async_copy(
                    tab_ref.at[idx_v.at[j]],
                    rows_v.at[pl.ds(j * _IDX_ROW, _IDX_ROW)],
                    sem,
                )
                for j in range(_K)
            ]
            for c in copies:
                c.wait()
            pltpu.sync_copy(rows_v, out_ref.at[pl.ds(row0 * _IDX_ROW, _CHUNK)])
            return _
        lax.fori_loop(0, n_chunks_per_w, body, 0)


def _gather_rows(tab, idx1, idx2):
    N = idx1.shape[0]
    H = tab.shape[1]
    assert N % (_NW * _CHUNK) == 0
    n_chunks_per_w = N // (_NW * _CHUNK)
    idx1 = idx1.reshape(N // _IDX_ROW, _IDX_ROW)
    idx2 = idx2.reshape(N // _IDX_ROW, _IDX_ROW)
    mesh = plsc.VectorSubcoreMesh(core_axis_name="c", subcore_axis_name="s")
    out_t = jax.ShapeDtypeStruct((N, H), jnp.float32)
    return pl.kernel(
        functools.partial(_gather_body, n_chunks_per_w),
        out_type=(out_t, out_t),
        mesh=mesh,
        scratch_types=[
            pltpu.VMEM((_K, _IDX_ROW), jnp.int32),
            pltpu.VMEM((_CHUNK, H), jnp.float32),
            pltpu.SemaphoreType.DMA,
        ],
        compiler_params=pltpu.CompilerParams(use_tc_tiling_on_sc=False),
    )(tab, idx1, idx2)


def kernel(sent1, sent2, emb, W):
    B, L = sent1.shape
    H = W.shape[0]
    embW = _project_table(emb, W)
    o1, o2 = _gather_rows(
        embW,
        sent1.reshape(-1).astype(jnp.int32),
        sent2.reshape(-1).astype(jnp.int32),
    )
    return o1.reshape(B, L, H), o2.reshape(B, L, H)


# speedup vs baseline: 1.6399x; 1.6399x over previous
import functools

import jax
import jax.numpy as jnp
from jax import lax
from jax.experimental import pallas as pl
from jax.experimental.pallas import tpu as pltpu
from jax.experimental.pallas import tpu_sc as plsc

_NC, _NS = 2, 16
_NW = _NC * _NS

_PB = 8192


def _proj_body(xa_ref, xb_ref, w_ref, o_ref):
    ya = lax.dot_general(xa_ref[...], w_ref[...], (((0,), (1,)), ((), ())),
                         preferred_element_type=jnp.float32)
    yb = lax.dot_general(xb_ref[...], w_ref[...], (((0,), (1,)), ((), ())),
                         preferred_element_type=jnp.float32)
    o_ref[...] = jnp.concatenate([ya, yb], axis=-1)


def _project_table(emb, W):
    V, D = emb.shape
    H = W.shape[0]
    n_steps = (V + 2 * _PB - 1) // (2 * _PB)
    packed = pl.pallas_call(
        _proj_body,
        grid=(n_steps,),
        in_specs=[
            pl.BlockSpec((D, _PB), lambda i: (0, 2 * i)),
            pl.BlockSpec(
                (D, _PB),
                lambda i: (0, jnp.minimum(2 * i + 1, (V - 1) // _PB)),
            ),
            pl.BlockSpec((H, D), lambda i: (0, 0)),
        ],
        out_specs=pl.BlockSpec((_PB, 2 * H), lambda i: (i, 0)),
        out_shape=jax.ShapeDtypeStruct((n_steps * _PB, 2 * H), jnp.float32),
    )(emb.T, emb.T, W)
    return packed.reshape(2 * n_steps * _PB, H)


def _permute_idx(idx):
    hi = idx // (2 * _PB)
    q = idx % (2 * _PB)
    return hi * (2 * _PB) + (q % _PB) * 2 + q // _PB


_CB = 8


def _gather_body(n_chunks_per_w, L, tab_ref, idx_ref, out_ref,
                 idx_v, rows_v, sem):
    wid = lax.axis_index("s") * _NC + lax.axis_index("c")
    b_base = wid * n_chunks_per_w * _CB

    def body(i, _):
        b0 = b_base + i * _CB
        pltpu.sync_copy(idx_ref.at[pl.ds(b0, _CB)], idx_v)
        copies = [
            pltpu.async_copy(tab_ref.at[idx_v.at[k]],
                             rows_v.at[pl.ds(k * L, L)], sem)
            for k in range(_CB)
        ]
        for c in copies:
            c.wait()
        pltpu.sync_copy(rows_v, out_ref.at[pl.ds(b0 * L, _CB * L)])
        return _

    lax.fori_loop(0, n_chunks_per_w, body, 0)


def _gather_rows(tab, idx):
    B, L = idx.shape
    H = tab.shape[1]
    assert B % (_NW * _CB) == 0
    n_chunks_per_w = B // (_NW * _CB)
    mesh = plsc.VectorSubcoreMesh(core_axis_name="c", subcore_axis_name="s")
    out_t = jax.ShapeDtypeStruct((B * L, H), jnp.float32)
    return pl.kernel(
        functools.partial(_gather_body, n_chunks_per_w, L),
        out_type=out_t,
        mesh=mesh,
        scratch_types=[
            pltpu.VMEM((_CB, L), jnp.int32),
            pltpu.VMEM((_CB * L, H), jnp.float32),
            pltpu.SemaphoreType.DMA,
        ],
        compiler_params=pltpu.CompilerParams(use_tc_tiling_on_sc=False),
    )(tab, idx)


def kernel(sent1, sent2, emb, W):
    B, L = sent1.shape
    H = W.shape[0]
    embW = _project_table(emb, W)
    outs = []
    for sent in (sent1, sent2):
        idx = _permute_idx(sent.astype(jnp.int32))
        g = _gather_rows(embW, idx)
        outs.append(g.reshape(B, L, H))
    return outs[0], outs[1]
